# baseline (speedup 1.0000x reference)
import jax
import jax.numpy as jnp
from jax.experimental import pallas as pl

F32 = jnp.float32
PREC = jax.lax.Precision.DEFAULT


def _mm(a, b):
    return jax.lax.dot_general(a, b, (((1,), (0,)), ((), ())),
                               preferred_element_type=F32, precision=PREC)


def _k1_body(x_ref, w_ref, b_ref, o_ref):
    o_ref[0] = jnp.maximum(_mm(x_ref[0], w_ref[...]) + b_ref[...], 0.0)


def _enc1(x, w1, b1):
    xp = jnp.pad(x, ((0, 0), (0, 0), (1, 1), (1, 1)))
    sls = [xp[:, :, di:di + 226:2, dj:dj + 226:2]
           for di in range(3) for dj in range(3)]
    X = jnp.stack(sls, axis=2)
    X = X.transpose(0, 3, 4, 1, 2).reshape(16, 12769, 27)
    X = jnp.pad(X, ((0, 0), (0, 7), (0, 5)))
    W = w1.transpose(1, 2, 3, 0).reshape(27, 192)
    W = jnp.pad(W, ((0, 5), (0, 0)))
    b = b1.reshape(1, 192)
    out = pl.pallas_call(
        _k1_body,
        grid=(16,),
        in_specs=[pl.BlockSpec((1, 12776, 32), lambda n: (n, 0, 0)),
                  pl.BlockSpec((32, 192), lambda n: (0, 0)),
                  pl.BlockSpec((1, 192), lambda n: (0, 0))],
        out_specs=pl.BlockSpec((1, 12776, 192), lambda n: (n, 0, 0)),
        out_shape=jax.ShapeDtypeStruct((16, 12776, 192), F32),
    )(X, W, b)
    return out[:, :12769].reshape(16, 113, 113, 192)


def _k2_body(p_ref, w_ref, b_ref, o_ref):
    acc = None
    t = 0
    for di in range(3):
        for dj in range(3):
            ph = (di % 2) * 2 + (dj % 2)
            off = (di // 2) * 58 + (dj // 2)
            m = _mm(p_ref[0, ph, pl.ds(off, 3368), :], w_ref[t])
            acc = m if acc is None else acc + m
            t += 1
    o_ref[0] = acc + b_ref[...]


def _enc2(h, w2, b2):
    hp = jnp.pad(h, ((0, 0), (1, 2), (1, 2), (0, 0)))
    P = hp.reshape(16, 58, 2, 58, 2, 192).transpose(0, 2, 4, 1, 3, 5)
    P = P.reshape(16, 4, 3364, 192)
    P = jnp.pad(P, ((0, 0), (0, 0), (0, 68), (0, 0)))
    W = w2.transpose(2, 3, 1, 0).reshape(9, 192, 64)
    b = b2.reshape(1, 64)
    out = pl.pallas_call(
        _k2_body,
        grid=(16,),
        in_specs=[pl.BlockSpec((1, 4, 3432, 192), lambda n: (n, 0, 0, 0)),
                  pl.BlockSpec((9, 192, 64), lambda n: (0, 0, 0)),
                  pl.BlockSpec((1, 64), lambda n: (0, 0))],
        out_specs=pl.BlockSpec((1, 3368, 64), lambda n: (n, 0, 0)),
        out_shape=jax.ShapeDtypeStruct((16, 3368, 64), F32),
    )(P, W, b)
    z58 = out[:, :3364].reshape(16, 58, 58, 64)[:, :57, :57, :]
    return z58


def _k3_body(x_ref, e_ref, q_ref, s_ref):
    x = x_ref[...]
    e = e_ref[...]
    e2 = jnp.sum(e * e, axis=1)
    x2 = jnp.sum(x * x, axis=1, keepdims=True)
    xe = jax.lax.dot_general(x, e, (((1,), (1,)), ((), ())),
                             preferred_element_type=F32, precision=PREC)
    d = x2 + e2[None, :] - 2.0 * xe
    idx = jnp.argmin(d, axis=1).astype(jnp.int32)
    oh = (idx[:, None] == jax.lax.broadcasted_iota(jnp.int32, (3264, 512), 1))
    q = _mm(oh.astype(F32), e)
    row = pl.program_id(0) * 3264 + jax.lax.broadcasted_iota(jnp.int32, (3264, 1), 0)
    m = (row < 51984).astype(F32)
    diff = (q - x) * m
    q_ref[...] = q
    s_ref[0] = jnp.full((8, 128), jnp.sum(diff * diff), F32)


def _vq(flat_x, emb):
    xpad = jnp.pad(flat_x, ((0, 240), (0, 0)))
    q, s = pl.pallas_call(
        _k3_body,
        grid=(16,),
        in_specs=[pl.BlockSpec((3264, 64), lambda i: (i, 0)),
                  pl.BlockSpec((512, 64), lambda i: (0, 0))],
        out_specs=[pl.BlockSpec((3264, 64), lambda i: (i, 0)),
                   pl.BlockSpec((1, 8, 128), lambda i: (i, 0, 0))],
        out_shape=[jax.ShapeDtypeStruct((52224, 64), F32),
                   jax.ShapeDtypeStruct((16, 8, 128), F32)],
    )(xpad, emb)
    return q[:51984], jnp.sum(s[:, 0, 0])


def _k4_body(q_ref, w_ref, b_ref, ee_ref, eo_ref, oe_ref, oo_ref):
    b = b_ref[...]

    def sl(o):
        return q_ref[0, pl.ds(o, 3256), :]

    ee = _mm(sl(0), w_ref[0])
    eo = _mm(sl(0), w_ref[1]) + _mm(sl(1), w_ref[2])
    oe = _mm(sl(0), w_ref[3]) + _mm(sl(57), w_ref[4])
    oo = (_mm(sl(0), w_ref[5]) + _mm(sl(1), w_ref[6])
          + _mm(sl(57), w_ref[7]) + _mm(sl(58), w_ref[8]))
    ee_ref[0] = jnp.maximum(ee + b, 0.0)
    eo_ref[0] = jnp.maximum(eo + b, 0.0)
    oe_ref[0] = jnp.maximum(oe + b, 0.0)
    oo_ref[0] = jnp.maximum(oo + b, 0.0)


def _dec1(qn, w, b):
    Q = qn.reshape(16, 3249, 64)
    Q = jnp.pad(Q, ((0, 0), (0, 71), (0, 0)))
    taps = [w[:, :, 1, 1],
            w[:, :, 1, 2], w[:, :, 1, 0],
            w[:, :, 2, 1], w[:, :, 0, 1],
            w[:, :, 2, 2], w[:, :, 2, 0], w[:, :, 0, 2], w[:, :, 0, 0]]
    W = jnp.stack(taps, axis=0)
    bb = b.reshape(1, 192)
    outs = pl.pallas_call(
        _k4_body,
        grid=(16,),
        in_specs=[pl.BlockSpec((1, 3320, 64), lambda n: (n, 0, 0)),
                  pl.BlockSpec((9, 64, 192), lambda n: (0, 0, 0)),
                  pl.BlockSpec((1, 192), lambda n: (0, 0))],
        out_specs=[pl.BlockSpec((1, 3256, 192), lambda n: (n, 0, 0))] * 4,
        out_shape=[jax.ShapeDtypeStruct((16, 3256, 192), F32)] * 4,
    )(Q, W, bb)
    ee, eo, oe, oo = [o[:, :3249].reshape(16, 57, 57, 192) for o in outs]
    evens = jnp.stack([ee, eo], axis=3).reshape(16, 57, 114, 192)[:, :, :113]
    odds = jnp.stack([oe, oo], axis=3).reshape(16, 57, 114, 192)[:, :, :113]
    d = jnp.stack([evens, odds], axis=2).reshape(16, 114, 113, 192)[:, :113]
    return d


def _k5_body(d_ref, w_ref, b_ref, xp_ref, m_ref, o_ref, s_ref):
    base = pl.multiple_of(pl.program_id(1) * 3200, 8)
    v = d_ref[0, pl.ds(base, 3320), :]

    def sl(o):
        return jax.lax.slice(v, (o, 0), (o + 3200, 192))

    acc = (_mm(sl(0), w_ref[0]) + _mm(sl(1), w_ref[1])
           + _mm(sl(113), w_ref[2]) + _mm(sl(114), w_ref[3]))
    sg = jax.nn.sigmoid(acc + b_ref[...])
    o_ref[0] = sg
    df = (sg - xp_ref[0]) * m_ref[...]
    s_ref[0, 0] = jnp.full((8, 128), jnp.sum(df * df), F32)


def _dec2(d, w, b, x):
    D = d.reshape(16, 12769, 192)
    D = jnp.pad(D, ((0, 0), (0, 159), (0, 0)))

    def tap(kh, kw):
        return jnp.pad(w[:, :, kh, kw], ((0, 0), (0, 5)))

    z = jnp.zeros((192, 8), F32)
    W0 = jnp.concatenate([tap(1, 1), tap(1, 2), tap(2, 1), tap(2, 2)], axis=1)
    W1 = jnp.concatenate([z, tap(1, 0), z, tap(2, 0)], axis=1)
    W113 = jnp.concatenate([z, z, tap(0, 1), tap(0, 2)], axis=1)
    W114 = jnp.concatenate([z, z, z, tap(0, 0)], axis=1)
    W = jnp.stack([W0, W1, W113, W114], axis=0)
    bb = jnp.tile(jnp.pad(b, (0, 5)), 4).reshape(1, 32)

    xr = x.transpose(0, 2, 3, 1)
    xee = xr[:, 0::2, 0::2]
    xeo = jnp.pad(xr[:, 0::2, 1::2], ((0, 0), (0, 0), (0, 1), (0, 0)))
    xoe = jnp.pad(xr[:, 1::2, 0::2], ((0, 0), (0, 1), (0, 0), (0, 0)))
    xoo = jnp.pad(xr[:, 1::2, 1::2], ((0, 0), (0, 1), (0, 1), (0, 0)))
    xph = jnp.stack([xee, xeo, xoe, xoo], axis=1).reshape(16, 4, 12769, 3)
    xph = jnp.pad(xph, ((0, 0), (0, 0), (0, 0), (0, 5)))
    xph = xph.transpose(0, 2, 1, 3).reshape(16, 12769, 32)
    xph = jnp.pad(xph, ((0, 0), (0, 31), (0, 0)))

    ii = jnp.arange(113)[:, None]
    jj = jnp.arange(113)[None, :]
    col_ok = jnp.broadcast_to(jj <= 111, (113, 113))
    row_ok = jnp.broadcast_to(ii <= 111, (113, 113))
    full = jnp.ones((113, 113), bool)
    mm = jnp.stack([full, col_ok, row_ok, row_ok & col_ok], 0).reshape(4, 12769)
    mask = (mm.T[:, :, None] & (jnp.arange(8) < 3)[None, None, :]).astype(F32)
    mask = jnp.pad(mask.reshape(12769, 32), ((0, 31), (0, 0)))

    sig, s = pl.pallas_call(
        _k5_body,
        grid=(16, 4),
        in_specs=[pl.BlockSpec((1, 12928, 192), lambda n, m: (n, 0, 0)),
                  pl.BlockSpec((4, 192, 32), lambda n, m: (0, 0, 0)),
                  pl.BlockSpec((1, 32), lambda n, m: (0, 0)),
                  pl.BlockSpec((1, 3200, 32), lambda n, m: (n, m, 0)),
                  pl.BlockSpec((3200, 32), lambda n, m: (m, 0))],
        out_specs=[pl.BlockSpec((1, 3200, 32), lambda n, m: (n, m, 0)),
                   pl.BlockSpec((1, 1, 8, 128), lambda n, m: (n, m, 0, 0))],
        out_shape=[jax.ShapeDtypeStruct((16, 12800, 32), F32),
                   jax.ShapeDtypeStruct((16, 4, 8, 128), F32)],
    )(D, W, bb, xph, mask)

    S = sig[:, :12769].reshape(16, 113, 113, 32)
    ph = [S[..., p * 8:p * 8 + 3] for p in range(4)]
    evens = jnp.stack([ph[0], ph[1]], axis=3).reshape(16, 113, 226, 3)[:, :, :225]
    odds = jnp.stack([ph[2], ph[3]], axis=3).reshape(16, 113, 226, 3)[:, :, :225]
    xr2 = jnp.stack([evens, odds], axis=2).reshape(16, 226, 225, 3)[:, :225]
    x_recon = xr2.transpose(0, 3, 1, 2)
    recon_sum = jnp.sum(s[:, :, 0, 0])
    return x_recon, recon_sum


def kernel(x, enc_w1, enc_b1, enc_w2, enc_b2, embedding,
           dec_w1, dec_b1, dec_w2, dec_b2):
    h = _enc1(x, enc_w1, enc_b1)
    z_nhwc = _enc2(h, enc_w2, enc_b2)
    z_nchw = z_nhwc.transpose(0, 3, 1, 2)
    flat_x = z_nchw.reshape(-1, 64)
    qflat, vq_sum = _vq(flat_x, embedding)
    qn = qflat.reshape(16, 64, 57, 57).transpose(0, 2, 3, 1)
    d = _dec1(qn, dec_w1, dec_b1)
    x_recon, recon_sum = _dec2(d, dec_w2, dec_b2, x)
    recon_loss = recon_sum / (16.0 * 3.0 * 225.0 * 225.0)
    vq_loss = 1.25 * vq_sum / 3326976.0
    return (x_recon, recon_loss + vq_loss)


# speedup vs baseline: 1.5675x; 1.5675x over previous
import jax
import jax.numpy as jnp
from jax.experimental import pallas as pl

F32 = jnp.float32
PREC = jax.lax.Precision.DEFAULT


def _mm(a, b):
    return jax.lax.dot_general(a, b, (((1,), (0,)), ((), ())),
                               preferred_element_type=F32, precision=PREC)


def _k1_body(x_ref, w_ref, b_ref, o_ref):
    p = pl.program_id(1)
    a = p // 2
    b = p % 2
    r = jax.lax.broadcasted_iota(jnp.int32, (3432, 1), 0)
    i = r // 58
    j = r - i * 58
    oh = 2 * i + a - 1
    ow = 2 * j + b - 1
    valid = ((oh >= 0) & (oh <= 112) & (ow >= 0) & (ow <= 112) & (r < 3364))
    y = jnp.maximum(_mm(x_ref[0, 0], w_ref[...]) + b_ref[...], 0.0)
    o_ref[0, 0] = jnp.where(valid, y, 0.0)


def _enc1(x, w1, b1):
    xb = jnp.pad(x, ((0, 0), (0, 0), (3, 5), (3, 5)))
    phases = []
    for a in range(2):
        for bb in range(2):
            sls = [xb[:, :, 2 * a + di:2 * a + di + 232:4,
                      2 * bb + dj:2 * bb + dj + 232:4]
                   for di in range(3) for dj in range(3)]
            ph = jnp.stack(sls, axis=2)
            ph = ph.transpose(0, 3, 4, 1, 2).reshape(16, 3364, 27)
            phases.append(ph)
    X = jnp.stack(phases, axis=1)
    X = jnp.pad(X, ((0, 0), (0, 0), (0, 68), (0, 5)))
    W = w1.transpose(1, 2, 3, 0).reshape(27, 192)
    W = jnp.pad(W, ((0, 5), (0, 0)))
    b = b1.reshape(1, 192)
    P = pl.pallas_call(
        _k1_body,
        grid=(16, 4),
        in_specs=[pl.BlockSpec((1, 1, 3432, 32), lambda n, p: (n, p, 0, 0)),
                  pl.BlockSpec((32, 192), lambda n, p: (0, 0)),
                  pl.BlockSpec((1, 192), lambda n, p: (0, 0))],
        out_specs=pl.BlockSpec((1, 1, 3432, 192), lambda n, p: (n, p, 0, 0)),
        out_shape=jax.ShapeDtypeStruct((16, 4, 3432, 192), F32),
    )(X, W, b)
    return P


def _k2_body(p_ref, w_ref, b_ref, o_ref):
    acc = None
    t = 0
    for di in range(3):
        for dj in range(3):
            ph = (di % 2) * 2 + (dj % 2)
            off = (di // 2) * 58 + (dj // 2)
            m = _mm(p_ref[0, ph, pl.ds(off, 3368), :], w_ref[t])
            acc = m if acc is None else acc + m
            t += 1
    o_ref[0] = acc + b_ref[...]


def _enc2(P, w2, b2):
    W = w2.transpose(2, 3, 1, 0).reshape(9, 192, 64)
    b = b2.reshape(1, 64)
    out = pl.pallas_call(
        _k2_body,
        grid=(16,),
        in_specs=[pl.BlockSpec((1, 4, 3432, 192), lambda n: (n, 0, 0, 0)),
                  pl.BlockSpec((9, 192, 64), lambda n: (0, 0, 0)),
                  pl.BlockSpec((1, 64), lambda n: (0, 0))],
        out_specs=pl.BlockSpec((1, 3368, 64), lambda n: (n, 0, 0)),
        out_shape=jax.ShapeDtypeStruct((16, 3368, 64), F32),
    )(P, W, b)
    z58 = out[:, :3364].reshape(16, 58, 58, 64)[:, :57, :57, :]
    return z58


def _k3_body(x_ref, e_ref, q_ref, s_ref):
    x = x_ref[...]
    e = e_ref[...]
    e2 = jnp.sum(e * e, axis=1)
    x2 = jnp.sum(x * x, axis=1, keepdims=True)
    xe = jax.lax.dot_general(x, e, (((1,), (1,)), ((), ())),
                             preferred_element_type=F32, precision=PREC)
    d = x2 + e2[None, :] - 2.0 * xe
    idx = jnp.argmin(d, axis=1).astype(jnp.int32)
    oh = (idx[:, None] == jax.lax.broadcasted_iota(jnp.int32, (3264, 512), 1))
    q = _mm(oh.astype(F32), e)
    row = pl.program_id(0) * 3264 + jax.lax.broadcasted_iota(jnp.int32, (3264, 1), 0)
    m = (row < 51984).astype(F32)
    diff = (q - x) * m
    q_ref[...] = q
    s_ref[0] = jnp.full((8, 128), jnp.sum(diff * diff), F32)


def _vq(flat_x, emb):
    xpad = jnp.pad(flat_x, ((0, 240), (0, 0)))
    q, s = pl.pallas_call(
        _k3_body,
        grid=(16,),
        in_specs=[pl.BlockSpec((3264, 64), lambda i: (i, 0)),
                  pl.BlockSpec((512, 64), lambda i: (0, 0))],
        out_specs=[pl.BlockSpec((3264, 64), lambda i: (i, 0)),
                   pl.BlockSpec((1, 8, 128), lambda i: (i, 0, 0))],
        out_shape=[jax.ShapeDtypeStruct((52224, 64), F32),
                   jax.ShapeDtypeStruct((16, 8, 128), F32)],
    )(xpad, emb)
    return q[:51984], jnp.sum(s[:, 0, 0])


def _k45_body(q_ref, w4_ref, b4_ref, w5_ref, b5_ref, xp_ref, m_ref,
              o_ref, s_ref):
    b4 = b4_ref[...]

    def sl4(o):
        return q_ref[0, pl.ds(o, 3328), :]

    ee = jnp.maximum(_mm(sl4(0), w4_ref[0]) + b4, 0.0)
    eo = jnp.maximum(_mm(sl4(0), w4_ref[1]) + _mm(sl4(1), w4_ref[2]) + b4, 0.0)
    oe = jnp.maximum(_mm(sl4(0), w4_ref[3]) + _mm(sl4(57), w4_ref[4]) + b4, 0.0)
    oo = jnp.maximum(_mm(sl4(0), w4_ref[5]) + _mm(sl4(1), w4_ref[6])
                     + _mm(sl4(57), w4_ref[7]) + _mm(sl4(58), w4_ref[8]), 0.0)
    planes = [ee, eo, oe, oo]

    acc = None
    t = 0
    for pr, du in ((0, 0), (1, 0), (0, 1)):
        for pc, dv in ((0, 0), (1, 0), (0, 1)):
            pln = planes[pr * 2 + pc]
            off = du * 57 + dv
            v = jax.lax.slice(pln, (off, 0), (off + 3264, 192))
            m = _mm(v, w5_ref[t])
            acc = m if acc is None else acc + m
            t += 1
    sg = jax.nn.sigmoid(acc + b5_ref[...])
    o_ref[0] = sg
    df = (sg - xp_ref[0]) * m_ref[...]
    s_ref[0] = jnp.full((8, 128), jnp.sum(df * df), F32)


def _dec(qn, w4, b4, w5, b5, x):
    Q = qn.reshape(16, 3249, 64)
    Q = jnp.pad(Q, ((0, 0), (0, 143), (0, 0)))
    taps4 = [w4[:, :, 1, 1],
             w4[:, :, 1, 2], w4[:, :, 1, 0],
             w4[:, :, 2, 1], w4[:, :, 0, 1],
             w4[:, :, 2, 2], w4[:, :, 2, 0], w4[:, :, 0, 2], w4[:, :, 0, 0]]
    W4 = jnp.stack(taps4, axis=0)
    bb4 = b4.reshape(1, 192)

    rterms = {(0, 0): [(0, 1), (1, 2)],
              (1, 0): [(1, 0), (2, 1), (3, 2)],
              (0, 1): [(3, 0)]}
    combos = []
    for pr, du in ((0, 0), (1, 0), (0, 1)):
        for pc, dv in ((0, 0), (1, 0), (0, 1)):
            blocks = []
            feed = {}
            for c, kh in rterms[(pr, du)]:
                for e, kw in rterms[(pc, dv)]:
                    feed[c * 4 + e] = (kh, kw)
            for blk in range(16):
                if blk in feed:
                    kh, kw = feed[blk]
                    blocks.append(jnp.pad(w5[:, :, kh, kw], ((0, 0), (0, 5))))
                else:
                    blocks.append(jnp.zeros((192, 8), F32))
            combos.append(jnp.concatenate(blocks, axis=1))
    W5 = jnp.stack(combos, axis=0)
    bb5 = jnp.tile(jnp.pad(b5, (0, 5)), 16).reshape(1, 128)

    xr = jnp.pad(x.transpose(0, 2, 3, 1), ((0, 0), (0, 3), (0, 3), (0, 0)))
    xp = xr.reshape(16, 57, 4, 57, 4, 3).transpose(0, 1, 3, 2, 4, 5)
    xp = jnp.pad(xp.reshape(16, 3249, 16, 3), ((0, 0), (0, 15), (0, 0), (0, 5)))
    xp = xp.reshape(16, 3264, 128)

    u_ok = (jnp.arange(57)[:, None] <=
            jnp.where(jnp.arange(4) == 0, 56, 55)[None, :])
    m6 = (u_ok[:, None, :, None, None] & u_ok[None, :, None, :, None]
          & (jnp.arange(8) < 3)[None, None, None, None, :])
    mask = jnp.pad(m6.reshape(3249, 128).astype(F32), ((0, 15), (0, 0)))

    rec, s = pl.pallas_call(
        _k45_body,
        grid=(16,),
        in_specs=[pl.BlockSpec((1, 3392, 64), lambda n: (n, 0, 0)),
                  pl.BlockSpec((9, 64, 192), lambda n: (0, 0, 0)),
                  pl.BlockSpec((1, 192), lambda n: (0, 0)),
                  pl.BlockSpec((9, 192, 128), lambda n: (0, 0, 0)),
                  pl.BlockSpec((1, 128), lambda n: (0, 0)),
                  pl.BlockSpec((1, 3264, 128), lambda n: (n, 0, 0)),
                  pl.BlockSpec((3264, 128), lambda n: (0, 0))],
        out_specs=[pl.BlockSpec((1, 3264, 128), lambda n: (n, 0, 0)),
                   pl.BlockSpec((1, 8, 128), lambda n: (n, 0, 0))],
        out_shape=[jax.ShapeDtypeStruct((16, 3264, 128), F32),
                   jax.ShapeDtypeStruct((16, 8, 128), F32)],
    )(Q, W4, bb4, W5, bb5, xp, mask)

    R = rec[:, :3249].reshape(16, 57, 57, 4, 4, 8)[..., :3]
    R = R.transpose(0, 1, 3, 2, 4, 5).reshape(16, 228, 228, 3)
    x_recon = R[:, :225, :225].transpose(0, 3, 1, 2)
    recon_sum = jnp.sum(s[:, 0, 0])
    return x_recon, recon_sum


def kernel(x, enc_w1, enc_b1, enc_w2, enc_b2, embedding,
           dec_w1, dec_b1, dec_w2, dec_b2):
    P = _enc1(x, enc_w1, enc_b1)
    z_nhwc = _enc2(P, enc_w2, enc_b2)
    z_nchw = z_nhwc.transpose(0, 3, 1, 2)
    flat_x = z_nchw.reshape(-1, 64)
    qflat, vq_sum = _vq(flat_x, embedding)
    qn = qflat.reshape(16, 64, 57, 57).transpose(0, 2, 3, 1)
    x_recon, recon_sum = _dec(qn, dec_w1, dec_b1, dec_w2, dec_b2, x)
    recon_loss = recon_sum / (16.0 * 3.0 * 225.0 * 225.0)
    vq_loss = 1.25 * vq_sum / 3326976.0
    return (x_recon, recon_loss + vq_loss)


# speedup vs baseline: 3.0564x; 1.9499x over previous
import jax
import jax.numpy as jnp
from jax.experimental import pallas as pl

F32 = jnp.float32
PREC = jax.lax.Precision.DEFAULT


def _mm(a, b):
    return jax.lax.dot_general(a, b, (((1,), (0,)), ((), ())),
                               preferred_element_type=F32, precision=PREC)


def _k1_body(x_ref, w_ref, b_ref, o_ref):
    p = pl.program_id(1)
    a = p // 2
    b = p % 2
    r = jax.lax.broadcasted_iota(jnp.int32, (3552, 1), 0)
    i = r // 59
    j = r - i * 59
    oh = 2 * i + a - 1
    ow = 2 * j + b - 1
    valid = ((oh >= 0) & (oh <= 112) & (ow >= 0) & (ow <= 112) & (r < 3481))
    y = jnp.maximum(_mm(x_ref[0, 0], w_ref[...]) + b_ref[...], 0.0)
    o_ref[0, 0] = jnp.where(valid, y, 0.0)


def _enc1(x, w1, b1):
    xb = jnp.pad(x.transpose(0, 2, 3, 1), ((0, 0), (3, 8), (3, 8), (0, 0)))
    Q4 = xb.reshape(16, 59, 4, 59, 4, 3).transpose(0, 2, 4, 1, 3, 5)

    def plane(u, v):
        pz = Q4[:, u % 4, v % 4]
        if u >= 4:
            pz = jnp.pad(pz[:, 1:], ((0, 0), (0, 1), (0, 0), (0, 0)))
        if v >= 4:
            pz = jnp.pad(pz[:, :, 1:], ((0, 0), (0, 0), (0, 1), (0, 0)))
        return pz

    phases = []
    for a in range(2):
        for bb in range(2):
            sls = [plane(2 * a + di, 2 * bb + dj)
                   for di in range(3) for dj in range(3)]
            ph = jnp.stack(sls, axis=-1).reshape(16, 59, 59, 27)
            phases.append(ph.reshape(16, 3481, 27))
    X = jnp.stack(phases, axis=1)
    X = jnp.pad(X, ((0, 0), (0, 0), (0, 71), (0, 5)))
    W = w1.transpose(1, 2, 3, 0).reshape(27, 192)
    W = jnp.pad(W, ((0, 5), (0, 0)))
    b = b1.reshape(1, 192)
    P = pl.pallas_call(
        _k1_body,
        grid=(16, 4),
        in_specs=[pl.BlockSpec((1, 1, 3552, 32), lambda n, p: (n, p, 0, 0)),
                  pl.BlockSpec((32, 192), lambda n, p: (0, 0)),
                  pl.BlockSpec((1, 192), lambda n, p: (0, 0))],
        out_specs=pl.BlockSpec((1, 1, 3552, 192), lambda n, p: (n, p, 0, 0)),
        out_shape=jax.ShapeDtypeStruct((16, 4, 3552, 192), F32),
    )(X, W, b)
    return P


def _k2_body(p_ref, w_ref, b_ref, o_ref):
    acc = None
    t = 0
    for di in range(3):
        for dj in range(3):
            ph = (di % 2) * 2 + (dj % 2)
            off = (di // 2) * 59 + (dj // 2)
            m = _mm(p_ref[0, ph, pl.ds(off, 3488), :], w_ref[t])
            acc = m if acc is None else acc + m
            t += 1
    o_ref[0] = acc + b_ref[...]


def _enc2(P, w2, b2):
    W = w2.transpose(2, 3, 1, 0).reshape(9, 192, 64)
    b = b2.reshape(1, 64)
    out = pl.pallas_call(
        _k2_body,
        grid=(16,),
        in_specs=[pl.BlockSpec((1, 4, 3552, 192), lambda n: (n, 0, 0, 0)),
                  pl.BlockSpec((9, 192, 64), lambda n: (0, 0, 0)),
                  pl.BlockSpec((1, 64), lambda n: (0, 0))],
        out_specs=pl.BlockSpec((1, 3488, 64), lambda n: (n, 0, 0)),
        out_shape=jax.ShapeDtypeStruct((16, 3488, 64), F32),
    )(P, W, b)
    z59 = out[:, :3481].reshape(16, 59, 59, 64)[:, :57, :57, :]
    return z59


def _k3_body(x_ref, e_ref, q_ref, s_ref):
    x = x_ref[...]
    e = e_ref[...]
    e2 = jnp.sum(e * e, axis=1)
    x2 = jnp.sum(x * x, axis=1, keepdims=True)
    xe = jax.lax.dot_general(x, e, (((1,), (1,)), ((), ())),
                             preferred_element_type=F32, precision=PREC)
    d = x2 + e2[None, :] - 2.0 * xe
    idx = jnp.argmin(d, axis=1).astype(jnp.int32)
    oh = (idx[:, None] == jax.lax.broadcasted_iota(jnp.int32, (3264, 512), 1))
    q = _mm(oh.astype(F32), e)
    row = pl.program_id(0) * 3264 + jax.lax.broadcasted_iota(jnp.int32, (3264, 1), 0)
    m = (row < 51984).astype(F32)
    diff = (q - x) * m
    q_ref[...] = q
    s_ref[0] = jnp.full((8, 128), jnp.sum(diff * diff), F32)


def _vq(flat_x, emb):
    xpad = jnp.pad(flat_x, ((0, 240), (0, 0)))
    q, s = pl.pallas_call(
        _k3_body,
        grid=(16,),
        in_specs=[pl.BlockSpec((3264, 64), lambda i: (i, 0)),
                  pl.BlockSpec((512, 64), lambda i: (0, 0))],
        out_specs=[pl.BlockSpec((3264, 64), lambda i: (i, 0)),
                   pl.BlockSpec((1, 8, 128), lambda i: (i, 0, 0))],
        out_shape=[jax.ShapeDtypeStruct((52224, 64), F32),
                   jax.ShapeDtypeStruct((16, 8, 128), F32)],
    )(xpad, emb)
    return q[:51984], jnp.sum(s[:, 0, 0])


def _k45_body(q_ref, w4_ref, b4_ref, w5_ref, b5_ref, xp_ref, m_ref,
              o_ref, s_ref):
    b4 = b4_ref[...]

    def sl4(o):
        return q_ref[0, pl.ds(o, 3328), :]

    ee = jnp.maximum(_mm(sl4(0), w4_ref[0]) + b4, 0.0)
    eo = jnp.maximum(_mm(sl4(0), w4_ref[1]) + _mm(sl4(1), w4_ref[2]) + b4, 0.0)
    oe = jnp.maximum(_mm(sl4(0), w4_ref[3]) + _mm(sl4(57), w4_ref[4]) + b4, 0.0)
    oo = jnp.maximum(_mm(sl4(0), w4_ref[5]) + _mm(sl4(1), w4_ref[6])
                     + _mm(sl4(57), w4_ref[7]) + _mm(sl4(58), w4_ref[8]), 0.0)
    planes = [ee, eo, oe, oo]

    acc = None
    t = 0
    for pr, du in ((0, 0), (1, 0), (0, 1)):
        for pc, dv in ((0, 0), (1, 0), (0, 1)):
            pln = planes[pr * 2 + pc]
            off = du * 57 + dv
            v = jax.lax.slice(pln, (off, 0), (off + 3264, 192))
            m = _mm(v, w5_ref[t])
            acc = m if acc is None else acc + m
            t += 1
    sg = jax.nn.sigmoid(acc + b5_ref[...])
    o_ref[0] = sg
    df = (sg - xp_ref[0]) * m_ref[...]
    s_ref[0] = jnp.full((8, 128), jnp.sum(df * df), F32)


def _dec(qn, w4, b4, w5, b5, x):
    Q = qn.reshape(16, 3249, 64)
    Q = jnp.pad(Q, ((0, 0), (0, 143), (0, 0)))
    taps4 = [w4[:, :, 1, 1],
             w4[:, :, 1, 2], w4[:, :, 1, 0],
             w4[:, :, 2, 1], w4[:, :, 0, 1],
             w4[:, :, 2, 2], w4[:, :, 2, 0], w4[:, :, 0, 2], w4[:, :, 0, 0]]
    W4 = jnp.stack(taps4, axis=0)
    bb4 = b4.reshape(1, 192)

    rterms = {(0, 0): [(0, 1), (1, 2)],
              (1, 0): [(1, 0), (2, 1), (3, 2)],
              (0, 1): [(3, 0)]}
    combos = []
    for pr, du in ((0, 0), (1, 0), (0, 1)):
        for pc, dv in ((0, 0), (1, 0), (0, 1)):
            blocks = []
            feed = {}
            for c, kh in rterms[(pr, du)]:
                for e, kw in rterms[(pc, dv)]:
                    feed[c * 4 + e] = (kh, kw)
            for blk in range(16):
                if blk in feed:
                    kh, kw = feed[blk]
                    blocks.append(jnp.pad(w5[:, :, kh, kw], ((0, 0), (0, 5))))
                else:
                    blocks.append(jnp.zeros((192, 8), F32))
            combos.append(jnp.concatenate(blocks, axis=1))
    W5 = jnp.stack(combos, axis=0)
    bb5 = jnp.tile(jnp.pad(b5, (0, 5)), 16).reshape(1, 128)

    xr = jnp.pad(x.transpose(0, 2, 3, 1), ((0, 0), (0, 3), (0, 3), (0, 0)))
    xp = xr.reshape(16, 57, 4, 57, 4, 3).transpose(0, 1, 3, 2, 4, 5)
    xp = jnp.pad(xp.reshape(16, 3249, 16, 3), ((0, 0), (0, 15), (0, 0), (0, 5)))
    xp = xp.reshape(16, 3264, 128)

    u_ok = (jnp.arange(57)[:, None] <=
            jnp.where(jnp.arange(4) == 0, 56, 55)[None, :])
    m6 = (u_ok[:, None, :, None, None] & u_ok[None, :, None, :, None]
          & (jnp.arange(8) < 3)[None, None, None, None, :])
    mask = jnp.pad(m6.reshape(3249, 128).astype(F32), ((0, 15), (0, 0)))

    rec, s = pl.pallas_call(
        _k45_body,
        grid=(16,),
        in_specs=[pl.BlockSpec((1, 3392, 64), lambda n: (n, 0, 0)),
                  pl.BlockSpec((9, 64, 192), lambda n: (0, 0, 0)),
                  pl.BlockSpec((1, 192), lambda n: (0, 0)),
                  pl.BlockSpec((9, 192, 128), lambda n: (0, 0, 0)),
                  pl.BlockSpec((1, 128), lambda n: (0, 0)),
                  pl.BlockSpec((1, 3264, 128), lambda n: (n, 0, 0)),
                  pl.BlockSpec((3264, 128), lambda n: (0, 0))],
        out_specs=[pl.BlockSpec((1, 3264, 128), lambda n: (n, 0, 0)),
                   pl.BlockSpec((1, 8, 128), lambda n: (n, 0, 0))],
        out_shape=[jax.ShapeDtypeStruct((16, 3264, 128), F32),
                   jax.ShapeDtypeStruct((16, 8, 128), F32)],
    )(Q, W4, bb4, W5, bb5, xp, mask)

    R = rec[:, :3249].reshape(16, 57, 57, 4, 4, 8)[..., :3]
    R = R.transpose(0, 1, 3, 2, 4, 5).reshape(16, 228, 228, 3)
    x_recon = R[:, :225, :225].transpose(0, 3, 1, 2)
    recon_sum = jnp.sum(s[:, 0, 0])
    return x_recon, recon_sum


def kernel(x, enc_w1, enc_b1, enc_w2, enc_b2, embedding,
           dec_w1, dec_b1, dec_w2, dec_b2):
    P = _enc1(x, enc_w1, enc_b1)
    z_nhwc = _enc2(P, enc_w2, enc_b2)
    z_nchw = z_nhwc.transpose(0, 3, 1, 2)
    flat_x = z_nchw.reshape(-1, 64)
    qflat, vq_sum = _vq(flat_x, embedding)
    qn = qflat.reshape(16, 64, 57, 57).transpose(0, 2, 3, 1)
    x_recon, recon_sum = _dec(qn, dec_w1, dec_b1, dec_w2, dec_b2, x)
    recon_loss = recon_sum / (16.0 * 3.0 * 225.0 * 225.0)
    vq_loss = 1.25 * vq_sum / 3326976.0
    return (x_recon, recon_loss + vq_loss)


# speedup vs baseline: 4.1771x; 1.3666x over previous
import jax
import jax.numpy as jnp
from jax.experimental import pallas as pl

F32 = jnp.float32
PREC = jax.lax.Precision.DEFAULT


def _mm(a, b):
    return jax.lax.dot_general(a, b, (((1,), (0,)), ((), ())),
                               preferred_element_type=F32, precision=PREC)


_K1_DELTAS = [[(0, 0)], [(0, 0), (0, 1)], [(0, 0), (1, 0)],
              [(0, 0), (0, 1), (1, 0), (1, 1)]]


def _k1_body(x_ref, w_ref, b_ref, o_ref):
    r = jax.lax.broadcasted_iota(jnp.int32, (3552, 1), 0)
    i = r // 59
    j = r - i * 59
    t = 0
    for p in range(4):
        a, b = p // 2, p % 2
        acc = None
        for ri, rj in _K1_DELTAS[p]:
            m = _mm(x_ref[0, pl.ds(ri * 59 + rj, 3552), :], w_ref[t])
            acc = m if acc is None else acc + m
            t += 1
        oh = 2 * i + a - 1
        ow = 2 * j + b - 1
        valid = ((oh >= 0) & (oh <= 112) & (ow >= 0) & (ow <= 112)
                 & (r < 3481))
        y = jnp.maximum(acc + b_ref[...], 0.0)
        o_ref[0, p] = jnp.where(valid, y, 0.0)


def _enc1(x, w1, b1):
    xb = jnp.pad(x.transpose(0, 2, 3, 1), ((0, 0), (3, 8), (3, 8), (0, 0)))
    Q4L = xb.reshape(16, 59, 4, 59, 4, 3).transpose(0, 1, 3, 2, 4, 5)
    Q4L = Q4L.reshape(16, 3481, 48)
    Q4L = jnp.pad(Q4L, ((0, 0), (0, 135), (0, 0)))

    blocks = []
    for p in range(4):
        a, b = p // 2, p % 2
        for ri, rj in _K1_DELTAS[p]:
            rows = []
            for u in range(4):
                for v in range(4):
                    di = u + 4 * ri - 2 * a
                    dj = v + 4 * rj - 2 * b
                    if 0 <= di <= 2 and 0 <= dj <= 2:
                        rows.append(w1[:, :, di, dj].T)
                    else:
                        rows.append(jnp.zeros((3, 192), F32))
            blocks.append(jnp.concatenate(rows, axis=0))
    W = jnp.stack(blocks, axis=0)
    b = b1.reshape(1, 192)
    P = pl.pallas_call(
        _k1_body,
        grid=(16,),
        in_specs=[pl.BlockSpec((1, 3616, 48), lambda n: (n, 0, 0)),
                  pl.BlockSpec((9, 48, 192), lambda n: (0, 0, 0)),
                  pl.BlockSpec((1, 192), lambda n: (0, 0))],
        out_specs=pl.BlockSpec((1, 4, 3552, 192), lambda n: (n, 0, 0, 0)),
        out_shape=jax.ShapeDtypeStruct((16, 4, 3552, 192), F32),
    )(Q4L, W, b)
    return P


def _k2_body(p_ref, w_ref, b_ref, o_ref):
    acc = None
    t = 0
    for di in range(3):
        for dj in range(3):
            ph = (di % 2) * 2 + (dj % 2)
            off = (di // 2) * 59 + (dj // 2)
            m = _mm(p_ref[0, ph, pl.ds(off, 3488), :], w_ref[t])
            acc = m if acc is None else acc + m
            t += 1
    o_ref[0] = acc + b_ref[...]


def _enc2(P, w2, b2):
    W = w2.transpose(2, 3, 1, 0).reshape(9, 192, 64)
    b = b2.reshape(1, 64)
    out = pl.pallas_call(
        _k2_body,
        grid=(16,),
        in_specs=[pl.BlockSpec((1, 4, 3552, 192), lambda n: (n, 0, 0, 0)),
                  pl.BlockSpec((9, 192, 64), lambda n: (0, 0, 0)),
                  pl.BlockSpec((1, 64), lambda n: (0, 0))],
        out_specs=pl.BlockSpec((1, 3488, 64), lambda n: (n, 0, 0)),
        out_shape=jax.ShapeDtypeStruct((16, 3488, 64), F32),
    )(P, W, b)
    z59 = out[:, :3481].reshape(16, 59, 59, 64)[:, :57, :57, :]
    return z59


def _k3_body(x_ref, e_ref, q_ref, s_ref):
    x = x_ref[...]
    e = e_ref[...]
    e2 = jnp.sum(e * e, axis=1)
    x2 = jnp.sum(x * x, axis=1, keepdims=True)
    xe = jax.lax.dot_general(x, e, (((1,), (1,)), ((), ())),
                             preferred_element_type=F32, precision=PREC)
    d = x2 + e2[None, :] - 2.0 * xe
    idx = jnp.argmin(d, axis=1).astype(jnp.int32)
    oh = (idx[:, None] == jax.lax.broadcasted_iota(jnp.int32, (3264, 512), 1))
    q = _mm(oh.astype(F32), e)
    row = pl.program_id(0) * 3264 + jax.lax.broadcasted_iota(jnp.int32, (3264, 1), 0)
    m = (row < 51984).astype(F32)
    diff = (q - x) * m
    q_ref[...] = q
    s_ref[0] = jnp.full((8, 128), jnp.sum(diff * diff), F32)


def _vq(flat_x, emb):
    xpad = jnp.pad(flat_x, ((0, 240), (0, 0)))
    q, s = pl.pallas_call(
        _k3_body,
        grid=(16,),
        in_specs=[pl.BlockSpec((3264, 64), lambda i: (i, 0)),
                  pl.BlockSpec((512, 64), lambda i: (0, 0))],
        out_specs=[pl.BlockSpec((3264, 64), lambda i: (i, 0)),
                   pl.BlockSpec((1, 8, 128), lambda i: (i, 0, 0))],
        out_shape=[jax.ShapeDtypeStruct((52224, 64), F32),
                   jax.ShapeDtypeStruct((16, 8, 128), F32)],
    )(xpad, emb)
    return q[:51984], jnp.sum(s[:, 0, 0])


def _k45_body(q_ref, w4_ref, b4_ref, w5_ref, b5_ref, xp_ref, m_ref,
              o_ref, s_ref):
    b4 = b4_ref[...]

    def sl4(o):
        return q_ref[0, pl.ds(o, 3328), :]

    ee = jnp.maximum(_mm(sl4(0), w4_ref[0]) + b4, 0.0)
    eo = jnp.maximum(_mm(sl4(0), w4_ref[1]) + _mm(sl4(1), w4_ref[2]) + b4, 0.0)
    oe = jnp.maximum(_mm(sl4(0), w4_ref[3]) + _mm(sl4(57), w4_ref[4]) + b4, 0.0)
    oo = jnp.maximum(_mm(sl4(0), w4_ref[5]) + _mm(sl4(1), w4_ref[6])
                     + _mm(sl4(57), w4_ref[7]) + _mm(sl4(58), w4_ref[8]), 0.0)
    planes = [ee, eo, oe, oo]

    acc = None
    t = 0
    for pr, du in ((0, 0), (1, 0), (0, 1)):
        for pc, dv in ((0, 0), (1, 0), (0, 1)):
            pln = planes[pr * 2 + pc]
            off = du * 57 + dv
            v = jax.lax.slice(pln, (off, 0), (off + 3264, 192))
            m = _mm(v, w5_ref[t])
            acc = m if acc is None else acc + m
            t += 1
    sg = jax.nn.sigmoid(acc + b5_ref[...])
    o_ref[0] = sg
    df = (sg - xp_ref[0]) * m_ref[...]
    s_ref[0] = jnp.full((8, 128), jnp.sum(df * df), F32)


def _dec(qn, w4, b4, w5, b5, x):
    Q = qn.reshape(16, 3249, 64)
    Q = jnp.pad(Q, ((0, 0), (0, 143), (0, 0)))
    taps4 = [w4[:, :, 1, 1],
             w4[:, :, 1, 2], w4[:, :, 1, 0],
             w4[:, :, 2, 1], w4[:, :, 0, 1],
             w4[:, :, 2, 2], w4[:, :, 2, 0], w4[:, :, 0, 2], w4[:, :, 0, 0]]
    W4 = jnp.stack(taps4, axis=0)
    bb4 = b4.reshape(1, 192)

    rterms = {(0, 0): [(0, 1), (1, 2)],
              (1, 0): [(1, 0), (2, 1), (3, 2)],
              (0, 1): [(3, 0)]}
    combos = []
    for pr, du in ((0, 0), (1, 0), (0, 1)):
        for pc, dv in ((0, 0), (1, 0), (0, 1)):
            blocks = []
            feed = {}
            for c, kh in rterms[(pr, du)]:
                for e, kw in rterms[(pc, dv)]:
                    feed[c * 4 + e] = (kh, kw)
            for blk in range(16):
                if blk in feed:
                    kh, kw = feed[blk]
                    blocks.append(jnp.pad(w5[:, :, kh, kw], ((0, 0), (0, 5))))
                else:
                    blocks.append(jnp.zeros((192, 8), F32))
            combos.append(jnp.concatenate(blocks, axis=1))
    W5 = jnp.stack(combos, axis=0)
    bb5 = jnp.tile(jnp.pad(b5, (0, 5)), 16).reshape(1, 128)

    xr = jnp.pad(x.transpose(0, 2, 3, 1), ((0, 0), (0, 3), (0, 3), (0, 0)))
    xp = xr.reshape(16, 57, 4, 57, 4, 3).transpose(0, 1, 3, 2, 4, 5)
    xp = jnp.pad(xp.reshape(16, 3249, 16, 3), ((0, 0), (0, 15), (0, 0), (0, 5)))
    xp = xp.reshape(16, 3264, 128)

    u_ok = (jnp.arange(57)[:, None] <=
            jnp.where(jnp.arange(4) == 0, 56, 55)[None, :])
    m6 = (u_ok[:, None, :, None, None] & u_ok[None, :, None, :, None]
          & (jnp.arange(8) < 3)[None, None, None, None, :])
    mask = jnp.pad(m6.reshape(3249, 128).astype(F32), ((0, 15), (0, 0)))

    rec, s = pl.pallas_call(
        _k45_body,
        grid=(16,),
        in_specs=[pl.BlockSpec((1, 3392, 64), lambda n: (n, 0, 0)),
                  pl.BlockSpec((9, 64, 192), lambda n: (0, 0, 0)),
                  pl.BlockSpec((1, 192), lambda n: (0, 0)),
                  pl.BlockSpec((9, 192, 128), lambda n: (0, 0, 0)),
                  pl.BlockSpec((1, 128), lambda n: (0, 0)),
                  pl.BlockSpec((1, 3264, 128), lambda n: (n, 0, 0)),
                  pl.BlockSpec((3264, 128), lambda n: (0, 0))],
        out_specs=[pl.BlockSpec((1, 3264, 128), lambda n: (n, 0, 0)),
                   pl.BlockSpec((1, 8, 128), lambda n: (n, 0, 0))],
        out_shape=[jax.ShapeDtypeStruct((16, 3264, 128), F32),
                   jax.ShapeDtypeStruct((16, 8, 128), F32)],
    )(Q, W4, bb4, W5, bb5, xp, mask)

    R = rec[:, :3249].reshape(16, 57, 57, 4, 4, 8)[..., :3]
    R = R.transpose(0, 1, 3, 2, 4, 5).reshape(16, 228, 228, 3)
    x_recon = R[:, :225, :225].transpose(0, 3, 1, 2)
    recon_sum = jnp.sum(s[:, 0, 0])
    return x_recon, recon_sum


def kernel(x, enc_w1, enc_b1, enc_w2, enc_b2, embedding,
           dec_w1, dec_b1, dec_w2, dec_b2):
    P = _enc1(x, enc_w1, enc_b1)
    z_nhwc = _enc2(P, enc_w2, enc_b2)
    z_nchw = z_nhwc.transpose(0, 3, 1, 2)
    flat_x = z_nchw.reshape(-1, 64)
    qflat, vq_sum = _vq(flat_x, embedding)
    qn = qflat.reshape(16, 64, 57, 57).transpose(0, 2, 3, 1)
    x_recon, recon_sum = _dec(qn, dec_w1, dec_b1, dec_w2, dec_b2, x)
    recon_loss = recon_sum / (16.0 * 3.0 * 225.0 * 225.0)
    vq_loss = 1.25 * vq_sum / 3326976.0
    return (x_recon, recon_loss + vq_loss)


# speedup vs baseline: 4.9061x; 1.1745x over previous
import jax
import jax.numpy as jnp
from jax.experimental import pallas as pl

F32 = jnp.float32
PREC = jax.lax.Precision.DEFAULT


def _mm(a, b):
    return jax.lax.dot_general(a, b, (((1,), (0,)), ((), ())),
                               preferred_element_type=F32, precision=PREC)


_K1_DELTAS = [[(0, 0)], [(0, 0), (0, 1)], [(0, 0), (1, 0)],
              [(0, 0), (0, 1), (1, 0), (1, 1)]]


def _kenc_body(x_ref, w_ref, b_ref, w2_ref, b2_ref, o_ref):
    r = jax.lax.broadcasted_iota(jnp.int32, (3552, 1), 0)
    i = r // 59
    j = r - i * 59
    t = 0
    planes = []
    for p in range(4):
        a, b = p // 2, p % 2
        acc = None
        for ri, rj in _K1_DELTAS[p]:
            m = _mm(x_ref[0, pl.ds(ri * 59 + rj, 3552), :], w_ref[t])
            acc = m if acc is None else acc + m
            t += 1
        oh = 2 * i + a - 1
        ow = 2 * j + b - 1
        valid = ((oh >= 0) & (oh <= 112) & (ow >= 0) & (ow <= 112)
                 & (r < 3481))
        y = jnp.maximum(acc + b_ref[...], 0.0)
        planes.append(jnp.where(valid, y, 0.0))
    acc = None
    t = 0
    for di in range(3):
        for dj in range(3):
            ph = (di % 2) * 2 + (dj % 2)
            off = (di // 2) * 59 + (dj // 2)
            v = jax.lax.slice(planes[ph], (off, 0), (off + 3488, 192))
            m = _mm(v, w2_ref[t])
            acc = m if acc is None else acc + m
            t += 1
    o_ref[0] = acc + b2_ref[...]


def _enc(x, w1, b1, w2, b2):
    xb = jnp.pad(x.transpose(0, 2, 3, 1), ((0, 0), (3, 8), (3, 8), (0, 0)))
    Q4L = xb.reshape(16, 59, 4, 59, 4, 3).transpose(0, 1, 3, 2, 4, 5)
    Q4L = Q4L.reshape(16, 3481, 48)
    Q4L = jnp.pad(Q4L, ((0, 0), (0, 135), (0, 0)))

    blocks = []
    for p in range(4):
        a, b = p // 2, p % 2
        for ri, rj in _K1_DELTAS[p]:
            rows = []
            for u in range(4):
                for v in range(4):
                    di = u + 4 * ri - 2 * a
                    dj = v + 4 * rj - 2 * b
                    if 0 <= di <= 2 and 0 <= dj <= 2:
                        rows.append(w1[:, :, di, dj].T)
                    else:
                        rows.append(jnp.zeros((3, 192), F32))
            blocks.append(jnp.concatenate(rows, axis=0))
    W = jnp.stack(blocks, axis=0)
    b = b1.reshape(1, 192)
    W2 = w2.transpose(2, 3, 1, 0).reshape(9, 192, 64)
    bb2 = b2.reshape(1, 64)
    out = pl.pallas_call(
        _kenc_body,
        grid=(16,),
        in_specs=[pl.BlockSpec((1, 3616, 48), lambda n: (n, 0, 0)),
                  pl.BlockSpec((9, 48, 192), lambda n: (0, 0, 0)),
                  pl.BlockSpec((1, 192), lambda n: (0, 0)),
                  pl.BlockSpec((9, 192, 64), lambda n: (0, 0, 0)),
                  pl.BlockSpec((1, 64), lambda n: (0, 0))],
        out_specs=pl.BlockSpec((1, 3488, 64), lambda n: (n, 0, 0)),
        out_shape=jax.ShapeDtypeStruct((16, 3488, 64), F32),
    )(Q4L, W, b, W2, bb2)
    z59 = out[:, :3481].reshape(16, 59, 59, 64)[:, :57, :57, :]
    return z59


def _k3_body(x_ref, e_ref, q_ref, s_ref):
    x = x_ref[...]
    e = e_ref[...]
    e2 = jnp.sum(e * e, axis=1)
    x2 = jnp.sum(x * x, axis=1, keepdims=True)
    xe = jax.lax.dot_general(x, e, (((1,), (1,)), ((), ())),
                             preferred_element_type=F32, precision=PREC)
    d = x2 + e2[None, :] - 2.0 * xe
    idx = jnp.argmin(d, axis=1).astype(jnp.int32)
    oh = (idx[:, None] == jax.lax.broadcasted_iota(jnp.int32, (3264, 512), 1))
    q = _mm(oh.astype(F32), e)
    row = pl.program_id(0) * 3264 + jax.lax.broadcasted_iota(jnp.int32, (3264, 1), 0)
    m = (row < 51984).astype(F32)
    dmin = jnp.min(d, axis=1, keepdims=True) * m
    q_ref[...] = q
    s_ref[0] = jnp.full((8, 128), jnp.sum(dmin), F32)


def _vq(flat_x, emb):
    xpad = jnp.pad(flat_x, ((0, 240), (0, 0)))
    q, s = pl.pallas_call(
        _k3_body,
        grid=(16,),
        in_specs=[pl.BlockSpec((3264, 64), lambda i: (i, 0)),
                  pl.BlockSpec((512, 64), lambda i: (0, 0))],
        out_specs=[pl.BlockSpec((3264, 64), lambda i: (i, 0)),
                   pl.BlockSpec((1, 8, 128), lambda i: (i, 0, 0))],
        out_shape=[jax.ShapeDtypeStruct((52224, 64), F32),
                   jax.ShapeDtypeStruct((16, 8, 128), F32)],
    )(xpad, emb)
    return q[:51984], jnp.sum(s[:, 0, 0])


def _k45_body(q_ref, w4_ref, b4_ref, w5_ref, b5_ref, o_ref):
    b4 = b4_ref[...]

    def sl4(o):
        return q_ref[0, pl.ds(o, 3328), :]

    ee = jnp.maximum(_mm(sl4(0), w4_ref[0]) + b4, 0.0)
    eo = jnp.maximum(_mm(sl4(0), w4_ref[1]) + _mm(sl4(1), w4_ref[2]) + b4, 0.0)
    oe = jnp.maximum(_mm(sl4(0), w4_ref[3]) + _mm(sl4(57), w4_ref[4]) + b4, 0.0)
    oo = jnp.maximum(_mm(sl4(0), w4_ref[5]) + _mm(sl4(1), w4_ref[6])
                     + _mm(sl4(57), w4_ref[7]) + _mm(sl4(58), w4_ref[8]), 0.0)
    planes = [ee, eo, oe, oo]

    acc = None
    t = 0
    for pr, du in ((0, 0), (1, 0), (0, 1)):
        for pc, dv in ((0, 0), (1, 0), (0, 1)):
            pln = planes[pr * 2 + pc]
            off = du * 57 + dv
            v = jax.lax.slice(pln, (off, 0), (off + 3264, 192))
            m = _mm(v, w5_ref[t])
            acc = m if acc is None else acc + m
            t += 1
    o_ref[0] = jax.nn.sigmoid(acc + b5_ref[...])


def _dec(qn, w4, b4, w5, b5, x):
    Q = qn.reshape(16, 3249, 64)
    Q = jnp.pad(Q, ((0, 0), (0, 143), (0, 0)))
    taps4 = [w4[:, :, 1, 1],
             w4[:, :, 1, 2], w4[:, :, 1, 0],
             w4[:, :, 2, 1], w4[:, :, 0, 1],
             w4[:, :, 2, 2], w4[:, :, 2, 0], w4[:, :, 0, 2], w4[:, :, 0, 0]]
    W4 = jnp.stack(taps4, axis=0)
    bb4 = b4.reshape(1, 192)

    rterms = {(0, 0): [(0, 1), (1, 2)],
              (1, 0): [(1, 0), (2, 1), (3, 2)],
              (0, 1): [(3, 0)]}
    combos = []
    for pr, du in ((0, 0), (1, 0), (0, 1)):
        for pc, dv in ((0, 0), (1, 0), (0, 1)):
            blocks = []
            feed = {}
            for c, kh in rterms[(pr, du)]:
                for e, kw in rterms[(pc, dv)]:
                    feed[c * 4 + e] = (kh, kw)
            for blk in range(16):
                if blk in feed:
                    kh, kw = feed[blk]
                    blocks.append(jnp.pad(w5[:, :, kh, kw], ((0, 0), (0, 5))))
                else:
                    blocks.append(jnp.zeros((192, 8), F32))
            combos.append(jnp.concatenate(blocks, axis=1))
    W5 = jnp.stack(combos, axis=0)
    bb5 = jnp.tile(jnp.pad(b5, (0, 5)), 16).reshape(1, 128)

    rec = pl.pallas_call(
        _k45_body,
        grid=(16,),
        in_specs=[pl.BlockSpec((1, 3392, 64), lambda n: (n, 0, 0)),
                  pl.BlockSpec((9, 64, 192), lambda n: (0, 0, 0)),
                  pl.BlockSpec((1, 192), lambda n: (0, 0)),
                  pl.BlockSpec((9, 192, 128), lambda n: (0, 0, 0)),
                  pl.BlockSpec((1, 128), lambda n: (0, 0))],
        out_specs=pl.BlockSpec((1, 3264, 128), lambda n: (n, 0, 0)),
        out_shape=jax.ShapeDtypeStruct((16, 3264, 128), F32),
    )(Q, W4, bb4, W5, bb5)

    R = rec[:, :3249].reshape(16, 57, 57, 4, 4, 8)[..., :3]
    R = R.transpose(0, 1, 3, 2, 4, 5).reshape(16, 228, 228, 3)
    x_recon = R[:, :225, :225].transpose(0, 3, 1, 2)
    return x_recon


def kernel(x, enc_w1, enc_b1, enc_w2, enc_b2, embedding,
           dec_w1, dec_b1, dec_w2, dec_b2):
    z_nhwc = _enc(x, enc_w1, enc_b1, enc_w2, enc_b2)
    z_nchw = z_nhwc.transpose(0, 3, 1, 2)
    flat_x = z_nchw.reshape(-1, 64)
    qflat, vq_sum = _vq(flat_x, embedding)
    qn = qflat.reshape(16, 64, 57, 57).transpose(0, 2, 3, 1)
    x_recon = _dec(qn, dec_w1, dec_b1, dec_w2, dec_b2, x)
    recon_loss = jnp.mean(jnp.square(x_recon - x))
    vq_loss = 1.25 * vq_sum / 3326976.0
    return (x_recon, recon_loss + vq_loss)
